# baseline (speedup 1.0000x reference)
import jax
import jax.numpy as jnp
from jax.experimental import pallas as pl

N = 4096
D = 128
BR = 512


def _gcn_norm_kernel(x_ref, adja_ref, adjb_ref, w_ref, out_ref):
    i = pl.program_id(0)
    h_blk = jnp.dot(x_ref[...], w_ref[...], preferred_element_type=jnp.float32)
    h16 = h_blk.astype(jnp.bfloat16)
    dims = (((0,), (0,)), ((), ()))
    part = jax.lax.dot_general(
        adja_ref[...].astype(jnp.bfloat16), h16[:BR],
        dimension_numbers=dims, preferred_element_type=jnp.float32,
    ) + jax.lax.dot_general(
        adjb_ref[...].astype(jnp.bfloat16), h16[BR:],
        dimension_numbers=dims, preferred_element_type=jnp.float32,
    )

    @pl.when(i == 0)
    def _init():
        out_ref[...] = part

    @pl.when(i > 0)
    def _accum():
        out_ref[...] += part

    @pl.when(i == pl.num_programs(0) - 1)
    def _finalize():
        a = out_ref[...]
        c = a - jnp.mean(a, axis=0, keepdims=True)
        rnorm = jnp.sqrt(1e-6 + jnp.sum(c * c, axis=1, keepdims=True))
        out_ref[...] = jnp.maximum(c / rnorm, 0.0)


def kernel(x, adj, W, b):
    del b
    out = pl.pallas_call(
        _gcn_norm_kernel,
        grid=(N // (2 * BR),),
        in_specs=[
            pl.BlockSpec((2 * BR, D), lambda i: (i, 0)),
            pl.BlockSpec((BR, N), lambda i: (2 * i, 0)),
            pl.BlockSpec((BR, N), lambda i: (2 * i + 1, 0)),
            pl.BlockSpec((D, D), lambda i: (0, 0)),
        ],
        out_specs=pl.BlockSpec((N, D), lambda i: (0, 0)),
        out_shape=jax.ShapeDtypeStruct((N, D), jnp.float32),
    )(x, adj, adj, W)
    return (out, adj)


# speedup vs baseline: 1.0417x; 1.0417x over previous
import jax
import jax.numpy as jnp
from jax.experimental import pallas as pl
from jax.experimental.pallas import tpu as pltpu

N = 4096
D = 128
BC = 512


def _gcn_norm_kernel(x_ref, adj_ref, w_ref, out_ref, h_ref):
    i = pl.program_id(0)

    @pl.when(i == 0)
    def _compute_h():
        h_ref[...] = jnp.dot(
            x_ref[...], w_ref[...], preferred_element_type=jnp.float32
        ).astype(jnp.bfloat16)

    part = jax.lax.dot_general(
        adj_ref[...].astype(jnp.bfloat16), h_ref[...],
        dimension_numbers=(((0,), (0,)), ((), ())),
        preferred_element_type=jnp.float32,
    )
    out_ref[pl.ds(i * BC, BC), :] = part

    @pl.when(i == pl.num_programs(0) - 1)
    def _finalize():
        a = out_ref[...]
        c = a - jnp.mean(a, axis=0, keepdims=True)
        rnorm = jnp.sqrt(1e-6 + jnp.sum(c * c, axis=1, keepdims=True))
        out_ref[...] = jnp.maximum(c / rnorm, 0.0)


def kernel(x, adj, W, b):
    del b
    out = pl.pallas_call(
        _gcn_norm_kernel,
        grid=(N // BC,),
        in_specs=[
            pl.BlockSpec((N, D), lambda i: (0, 0)),
            pl.BlockSpec((N, BC), lambda i: (0, i)),
            pl.BlockSpec((D, D), lambda i: (0, 0)),
        ],
        out_specs=pl.BlockSpec((N, D), lambda i: (0, 0)),
        out_shape=jax.ShapeDtypeStruct((N, D), jnp.float32),
        scratch_shapes=[pltpu.VMEM((N, D), jnp.bfloat16)],
    )(x, adj, W)
    return (out, adj)
